# baseline (speedup 1.0000x reference)
import functools

import jax
import jax.numpy as jnp
from jax import lax
from jax.experimental import pallas as pl
from jax.experimental.pallas import tpu as pltpu

N_OUT = 4096
N_IN = 4096
D = 4
K = 256

BR = 8
GRID = N_OUT // BR


def _tc_body(cb_ref, idx_ref, out_ref):
    a = idx_ref[...]
    hi = a >= 128
    al = a & 127
    pat = lax.broadcasted_iota(jnp.int32, (BR, 128), 1)
    pattern = (pat % 4) * 32 + pat // 4
    planes = []
    for j in range(D):
        col = cb_ref[j, :]
        tlo = jnp.broadcast_to(col[:128], (BR, 128))
        thi = jnp.broadcast_to(col[128:], (BR, 128))
        glo = jnp.take_along_axis(tlo, al, axis=1)
        ghi = jnp.take_along_axis(thi, al, axis=1)
        planes.append(jnp.where(hi, ghi, glo))
    for g in range(N_IN // 128):
        parts = [
            lax.slice(planes[j], (0, g * 32), (BR, g * 32 + 32))
            for j in range(D)
        ]
        cat = jnp.concatenate(parts, axis=1)
        out_ref[:, g * 128:(g + 1) * 128] = jnp.take_along_axis(
            cat, pattern, axis=1)


_tc_gather = pl.pallas_call(
    _tc_body,
    out_shape=jax.ShapeDtypeStruct((N_OUT, N_IN), jnp.float32),
    grid=(GRID,),
    in_specs=[
        pl.BlockSpec((8, K), lambda i: (0, 0)),
        pl.BlockSpec((BR, N_IN // D), lambda i: (i, 0)),
    ],
    out_specs=pl.BlockSpec((BR, N_IN), lambda i: (i, 0)),
)


def kernel(centriods, assignments):
    cbT = jnp.zeros((8, K), jnp.float32).at[:D, :].set(centriods.T)
    a2 = assignments.reshape(N_OUT, N_IN // D)
    return _tc_gather(cbT, a2)


# speedup vs baseline: 3.2694x; 3.2694x over previous
import functools

import jax
import jax.numpy as jnp
from jax import lax
from jax.experimental import pallas as pl
from jax.experimental.pallas import tpu as pltpu
from jax.experimental.pallas import tpu_sc as plsc

N_OUT = 4096
N_IN = 4096
D = 4
K = 256
NUM_IDX = N_OUT * N_IN // D

NC = 2
NS = 16
CHUNK = 4096
SC_CHUNK = NS * CHUNK
NCHUNK = NUM_IDX // NC // SC_CHUNK
OUT_W = CHUNK * D
SC_OUT_W = SC_CHUNK * D


def _body(cb_hbm, idx_hbm, out_hbm,
          cb_v, idx_v0, idx_v1, outd_v0, outd_v1, outs_v0, outs_v1,
          shared0, shared1,
          cb_sem, in_sem0, in_sem1, od_sem0, od_sem1, xb_sem0, xb_sem1,
          big_sem0, big_sem1):
    c = lax.axis_index("c")
    s = lax.axis_index("s")
    sc_base = c * (NUM_IDX // NC)

    pltpu.async_copy(cb_hbm, cb_v, cb_sem).wait()

    idx_bufs = (idx_v0, idx_v1)
    outd_bufs = (outd_v0, outd_v1)
    outs_bufs = (outs_v0, outs_v1)
    shareds = (shared0, shared1)
    in_sems = (in_sem0, in_sem1)
    od_sems = (od_sem0, od_sem1)
    xb_sems = (xb_sem0, xb_sem1)
    big_sems = (big_sem0, big_sem1)

    lane = lax.iota(jnp.int32, 16)
    st_base = lane * 4

    def start_in(g):
        b = g % 2
        return pltpu.async_copy(
            idx_hbm.at[pl.ds(sc_base + g * SC_CHUNK + s * CHUNK, CHUNK)],
            idx_bufs[b], in_sems[b])

    def start_direct_out(g, de):
        return pltpu.async_copy(
            outd_bufs[de],
            out_hbm.at[pl.ds((sc_base + g * SC_CHUNK + s * CHUNK) * D, OUT_W)],
            od_sems[de])

    def start_xb(sb):
        return pltpu.async_copy(
            outs_bufs[sb], shareds[sb].at[pl.ds(s * OUT_W, OUT_W)], xb_sems[sb])

    def big_desc(sidx):
        sb = sidx % 2
        g = 2 * sidx + 1
        return pltpu.make_async_copy(
            shareds[sb],
            out_hbm.at[pl.ds((sc_base + g * SC_CHUNK) * D, SC_OUT_W)],
            big_sems[sb])

    def compute(idx_ref, out_ref):
        @plsc.parallel_loop(0, CHUNK // 16, unroll=1)
        def body(i):
            a = idx_ref[pl.ds(i * 16, 16)]
            w = a * 4
            ob = i * 64
            vals = [plsc.load_gather(cb_v, [w + j]) for j in range(D)]
            for j in range(D):
                plsc.store_scatter(out_ref, [st_base + (ob + j)], vals[j])

    in_copies = [None, None]
    od_copies = [None, None]
    xb_copies = [None, None]
    in_copies[0] = start_in(0)
    for g in range(NCHUNK):
        b = g % 2
        if g + 1 < NCHUNK:
            in_copies[1 - b] = start_in(g + 1)
        in_copies[b].wait()
        if b == 0:
            de = (g // 2) % 2
            if od_copies[de] is not None:
                od_copies[de].wait()
            compute(idx_bufs[b], outd_bufs[de])
            od_copies[de] = start_direct_out(g, de)
        else:
            sidx = g // 2
            sb = sidx % 2
            if sidx >= 1:
                xb_copies[1 - sb].wait()
                plsc.subcore_barrier()

                @pl.when(s == 0)
                def _():
                    big_desc(sidx - 1).start()
            if sidx >= 2:
                @pl.when(s == 0)
                def _():
                    big_desc(sidx - 2).wait()
                plsc.subcore_barrier()
            compute(idx_bufs[b], outs_bufs[sb])
            xb_copies[sb] = start_xb(sb)
    last = NCHUNK // 2 - 1
    xb_copies[last % 2].wait()
    plsc.subcore_barrier()

    @pl.when(s == 0)
    def _():
        big_desc(last).start()
        big_desc(last - 1).wait()
        big_desc(last).wait()
    od_copies[0].wait()
    od_copies[1].wait()


_gather = functools.partial(
    pl.kernel,
    out_type=jax.ShapeDtypeStruct((NUM_IDX * D,), jnp.float32),
    mesh=plsc.VectorSubcoreMesh(core_axis_name="c", subcore_axis_name="s"),
    compiler_params=pltpu.CompilerParams(needs_layout_passes=False),
    scratch_types=[
        pltpu.VMEM((K * D,), jnp.float32),
        pltpu.VMEM((CHUNK,), jnp.int32),
        pltpu.VMEM((CHUNK,), jnp.int32),
        pltpu.VMEM((OUT_W,), jnp.float32),
        pltpu.VMEM((OUT_W,), jnp.float32),
        pltpu.VMEM((OUT_W,), jnp.float32),
        pltpu.VMEM((OUT_W,), jnp.float32),
        pltpu.VMEM_SHARED((SC_OUT_W,), jnp.float32),
        pltpu.VMEM_SHARED((SC_OUT_W,), jnp.float32),
        pltpu.SemaphoreType.DMA,
        pltpu.SemaphoreType.DMA,
        pltpu.SemaphoreType.DMA,
        pltpu.SemaphoreType.DMA,
        pltpu.SemaphoreType.DMA,
        pltpu.SemaphoreType.DMA,
        pltpu.SemaphoreType.DMA,
        pltpu.SemaphoreType.DMA,
        pltpu.SemaphoreType.DMA,
    ],
)(_body)


def kernel(centriods, assignments):
    out_flat = _gather(centriods.reshape(K * D), assignments)
    return out_flat.reshape(N_OUT, N_IN)


# speedup vs baseline: 3.4262x; 1.0480x over previous
import functools

import jax
import jax.numpy as jnp
from jax import lax
from jax.experimental import pallas as pl
from jax.experimental.pallas import tpu as pltpu
from jax.experimental.pallas import tpu_sc as plsc

N_OUT = 4096
N_IN = 4096
D = 4
K = 256
NUM_IDX = N_OUT * N_IN // D

NC = 2
NS = 16
NW = NC * NS
IDX_PER_W = NUM_IDX // NW
CHUNK = 8192
NCHUNK = IDX_PER_W // CHUNK


def _body(cb_hbm, idx_hbm, out_hbm,
          cb_v, idx_v0, idx_v1, out_v0, out_v1,
          cb_sem, in_sem0, in_sem1, out_sem0, out_sem1):
    wid = lax.axis_index("s") * NC + lax.axis_index("c")
    base = wid * IDX_PER_W

    pltpu.async_copy(cb_hbm, cb_v, cb_sem).wait()

    idx_bufs = (idx_v0, idx_v1)
    out_bufs = (out_v0, out_v1)
    in_sems = (in_sem0, in_sem1)
    out_sems = (out_sem0, out_sem1)

    lane = lax.iota(jnp.int32, 16)
    st_base = lane * 4

    def start_in(g):
        b = g % 2
        return pltpu.async_copy(
            idx_hbm.at[pl.ds(base + g * CHUNK, CHUNK)], idx_bufs[b], in_sems[b])

    def start_out(g):
        b = g % 2
        return pltpu.async_copy(
            out_bufs[b], out_hbm.at[pl.ds((base + g * CHUNK) * D, CHUNK * D)],
            out_sems[b])

    def compute(idx_ref, out_ref):
        @plsc.parallel_loop(0, CHUNK // 16, unroll=1)
        def body(i):
            a = idx_ref[pl.ds(i * 16, 16)]
            w = a * 4
            ob = i * 64
            vals = [plsc.load_gather(cb_v, [w + j]) for j in range(D)]
            for j in range(D):
                plsc.store_scatter(out_ref, [st_base + (ob + j)], vals[j])

    in_copies = [None, None]
    out_copies = [None, None]
    in_copies[0] = start_in(0)
    for g in range(NCHUNK):
        b = g % 2
        if g + 1 < NCHUNK:
            in_copies[1 - b] = start_in(g + 1)
        in_copies[b].wait()
        if out_copies[b] is not None:
            out_copies[b].wait()
        compute(idx_bufs[b], out_bufs[b])
        out_copies[b] = start_out(g)
    out_copies[0].wait()
    out_copies[1].wait()


_gather = functools.partial(
    pl.kernel,
    out_type=jax.ShapeDtypeStruct((NUM_IDX * D,), jnp.float32),
    mesh=plsc.VectorSubcoreMesh(core_axis_name="c", subcore_axis_name="s"),
    compiler_params=pltpu.CompilerParams(needs_layout_passes=False),
    scratch_types=[
        pltpu.VMEM((K * D,), jnp.float32),
        pltpu.VMEM((CHUNK,), jnp.int32),
        pltpu.VMEM((CHUNK,), jnp.int32),
        pltpu.VMEM((CHUNK * D,), jnp.float32),
        pltpu.VMEM((CHUNK * D,), jnp.float32),
        pltpu.SemaphoreType.DMA,
        pltpu.SemaphoreType.DMA,
        pltpu.SemaphoreType.DMA,
        pltpu.SemaphoreType.DMA,
        pltpu.SemaphoreType.DMA,
    ],
)(_body)


def kernel(centriods, assignments):
    out_flat = _gather(centriods.reshape(K * D), assignments)
    return out_flat.reshape(N_OUT, N_IN)
